# Initial kernel scaffold; baseline (speedup 1.0000x reference)
import jax
import jax.numpy as jnp
from jax.experimental import pallas as pl


def kernel(x, adj, encW, encB, encA, encG, encBeta, decW, decB, decA, decG, decBeta):
    raise NotImplementedError("write your pallas kernel here")


# speedup vs baseline: 1.0912x; 1.0912x over previous
import jax
import jax.numpy as jnp
from jax.experimental import pallas as pl
from jax.experimental.pallas import tpu as pltpu

_N = 8192
_D = 32
_L = 5
_BR = 512
_BR_CAST = 256
_EPS = 1e-5


def _cast_body(adj_ref, out_ref):
    out_ref[...] = adj_ref[...].astype(jnp.bfloat16)


def _layer_body(adj_ref, fts_ref, b_ref, a_ref, out_ref):
    f = fts_ref[...]
    hi = f.astype(jnp.bfloat16)
    lo = (f - hi.astype(jnp.float32)).astype(jnp.bfloat16)
    yy = jnp.dot(adj_ref[...], jnp.concatenate([hi, lo], axis=1),
                 preferred_element_type=jnp.float32)
    y = yy[:, :_D] + yy[:, _D:] + b_ref[...]
    out_ref[...] = jnp.where(y >= 0, y, a_ref[...] * y)


def _gcn_layer(adj_bf, fts, b, a_row):
    return pl.pallas_call(
        _layer_body,
        grid=(_N // _BR,),
        in_specs=[
            pl.BlockSpec((_BR, _N), lambda i: (i, 0)),
            pl.BlockSpec((_N, _D), lambda i: (0, 0)),
            pl.BlockSpec((1, _D), lambda i: (0, 0)),
            pl.BlockSpec((1, _D), lambda i: (0, 0)),
        ],
        out_specs=pl.BlockSpec((_BR, _D), lambda i: (i, 0)),
        out_shape=jax.ShapeDtypeStruct((_N, _D), jnp.float32),
        compiler_params=pltpu.CompilerParams(
            dimension_semantics=("arbitrary",)),
    )(adj_bf, fts, b, a_row)


def _bn(xx, g, b):
    m = xx.mean(axis=0, keepdims=True)
    v = xx.var(axis=0, keepdims=True)
    return (xx - m) / jnp.sqrt(v + _EPS) * g + b


def kernel(x, adj, encW, encB, encA, encG, encBeta,
           decW, decB, decA, decG, decBeta):
    adj_bf = pl.pallas_call(
        _cast_body,
        grid=(_N // _BR_CAST,),
        in_specs=[pl.BlockSpec((_BR_CAST, _N), lambda i: (i, 0))],
        out_specs=pl.BlockSpec((_BR_CAST, _N), lambda i: (i, 0)),
        out_shape=jax.ShapeDtypeStruct((_N, _N), jnp.bfloat16),
        compiler_params=pltpu.CompilerParams(
            dimension_semantics=("arbitrary",)),
    )(adj[0])

    h = x
    for i in range(_L):
        fts = h @ encW[i]
        a_row = jnp.broadcast_to(encA[i][None, None], (1, _D))
        out = _gcn_layer(adj_bf, fts[0], encB[i][None], a_row)[None]
        h = _bn(out[0], encG[i], encBeta[i])[None]
    encoded = h
    d = h
    for i in range(_L):
        fts = d @ decW[i]
        a_row = jnp.broadcast_to(decA[i][None, None], (1, _D))
        out = _gcn_layer(adj_bf, fts[0], decB[i][None], a_row)[None]
        if i < _L - 1:
            d = _bn(out[0], decG[i], decBeta[i])[None]
        else:
            d = out
    return (x, encoded, d)


# speedup vs baseline: 1.3137x; 1.2038x over previous
import jax
import jax.numpy as jnp
from jax.experimental import pallas as pl
from jax.experimental.pallas import tpu as pltpu

_N = 8192
_D = 32
_L = 5
_NL = 2 * _L
_BR = 512
_NB = _N // _BR
_EPS = 1e-5
_BR_CAST = 256


def _cast_body(adj_ref, out_ref):
    out_ref[...] = adj_ref[...].astype(jnp.bfloat16)


def _layers_body(x_ref, adj_ref, w_ref, vec_ref,
                 enc_out_ref, d_out_ref,
                 h_ref, enc_ref, fts_ref, stats_ref):
    l = pl.program_id(0)
    b = pl.program_id(1)

    @pl.when(b == 0)
    def _prologue():
        @pl.when(l == 0)
        def _init_h():
            h_ref[...] = x_ref[...]

        g = vec_ref[0, 2:3, :]
        beta = vec_ref[0, 3:4, :]
        m = stats_ref[0:1, :] * (1.0 / _N)
        v = stats_ref[1:2, :] * (1.0 / _N) - m * m
        s = g * jax.lax.rsqrt(v + _EPS)
        t = beta - m * s
        s = jnp.where(l == 0, jnp.ones_like(s), s)
        t = jnp.where(l == 0, jnp.zeros_like(t), t)
        ha = h_ref[...] * s + t

        @pl.when(l == _L)
        def _store_encoded():
            enc_ref[...] = ha

        fts = jnp.dot(ha.astype(jnp.bfloat16),
                      w_ref[0].astype(jnp.bfloat16),
                      preferred_element_type=jnp.float32)
        fts_ref[...] = fts.astype(jnp.bfloat16)
        stats_ref[...] = jnp.zeros_like(stats_ref)

    bias = vec_ref[0, 0:1, :]
    alpha = vec_ref[0, 1:2, :]
    y = jnp.dot(adj_ref[...], fts_ref[...],
                preferred_element_type=jnp.float32) + bias
    p = jnp.where(y >= 0, y, alpha * y)
    stats_ref[0:1, :] += jnp.sum(p, axis=0, keepdims=True)
    stats_ref[1:2, :] += jnp.sum(p * p, axis=0, keepdims=True)
    h_ref[pl.ds(b * _BR, _BR), :] = p
    d_out_ref[...] = p
    enc_out_ref[...] = enc_ref[pl.ds(b * _BR, _BR), :]


def kernel(x, adj, encW, encB, encA, encG, encBeta,
           decW, decB, decA, decG, decBeta):
    x2 = x[0]
    adj2 = adj[0]

    adj_bf = pl.pallas_call(
        _cast_body,
        grid=(_N // _BR_CAST,),
        in_specs=[pl.BlockSpec((_BR_CAST, _N), lambda i: (i, 0))],
        out_specs=pl.BlockSpec((_BR_CAST, _N), lambda i: (i, 0)),
        out_shape=jax.ShapeDtypeStruct((_N, _N), jnp.bfloat16),
        compiler_params=pltpu.CompilerParams(
            dimension_semantics=("arbitrary",)),
    )(adj2)

    w_all = jnp.concatenate([encW, decW], axis=0)
    b_all = jnp.concatenate([encB, decB], axis=0)
    a_all = jnp.broadcast_to(
        jnp.concatenate([encA, decA], axis=0)[:, None], (_NL, _D))
    g_fold = jnp.concatenate(
        [jnp.ones((1, _D), jnp.float32), encG, decG], axis=0)
    beta_fold = jnp.concatenate(
        [jnp.zeros((1, _D), jnp.float32), encBeta, decBeta], axis=0)
    vecs = jnp.stack([b_all, a_all, g_fold, beta_fold], axis=1)

    enc_out, d_out = pl.pallas_call(
        _layers_body,
        grid=(_NL, _NB),
        in_specs=[
            pl.BlockSpec((_N, _D), lambda l, b: (0, 0)),
            pl.BlockSpec((_BR, _N), lambda l, b: (b, 0)),
            pl.BlockSpec((1, _D, _D), lambda l, b: (l, 0, 0)),
            pl.BlockSpec((1, 4, _D), lambda l, b: (l, 0, 0)),
        ],
        out_specs=[
            pl.BlockSpec((_BR, _D), lambda l, b: (b, 0)),
            pl.BlockSpec((_BR, _D), lambda l, b: (b, 0)),
        ],
        out_shape=[jax.ShapeDtypeStruct((_N, _D), jnp.float32),
                   jax.ShapeDtypeStruct((_N, _D), jnp.float32)],
        scratch_shapes=[
            pltpu.VMEM((_N, _D), jnp.float32),
            pltpu.VMEM((_N, _D), jnp.float32),
            pltpu.VMEM((_N, _D), jnp.bfloat16),
            pltpu.VMEM((2, _D), jnp.float32),
        ],
        compiler_params=pltpu.CompilerParams(
            dimension_semantics=("arbitrary", "arbitrary")),
    )(x2, adj_bf, w_all, vecs)

    return (x, enc_out[None], d_out[None])
